# baseline (speedup 1.0000x reference)
import jax
import jax.numpy as jnp
from jax import lax
from jax.experimental import pallas as pl
from jax.experimental.pallas import tpu as pltpu
from jax.experimental.pallas import tpu_sc as plsc

NC, NS, L = 2, 16, 16
NW = NC * NS
ROWS, COLS = 128, 32768
RPW = ROWS // NW
NVEC = COLS // L
CVEC = 16
NCHUNK = NVEC // CVEC
NGRP = NCHUNK // L


def _insert5(ms, v):
    out = []
    for i in range(4):
        hi = jnp.maximum(ms[i], v)
        v = jnp.minimum(ms[i], v)
        out.append(hi)
    out.append(jnp.maximum(ms[4], v))
    return out


def _merge5(a, b):
    r = []
    for k in range(5):
        cur = jnp.maximum(a[k], b[k])
        for i in range(k):
            cur = jnp.maximum(cur, jnp.minimum(a[i], b[k - 1 - i]))
        r.append(cur)
    return r


def _splat(val):
    return jnp.full((L,), val, dtype=jnp.int32)


def _body(logits_ref, w1_ref, b1_ref, w2_ref, b2_ref, out_ref,
          buf0, buf1, cmbuf, idxbuf, cand, w1v, b1v, w2v, b2v, obuf,
          sem0, sem1, wsem):
    wid = lax.axis_index("s") * NC + lax.axis_index("c")
    base = wid * RPW
    bufs = [buf0, buf1]
    sems = [sem0, sem1]

    wc = [pltpu.async_copy(w1_ref, w1v, wsem),
          pltpu.async_copy(b1_ref, b1v, wsem),
          pltpu.async_copy(w2_ref, w2v, wsem),
          pltpu.async_copy(b2_ref, b2v, wsem)]
    descs = [
        pltpu.async_copy(logits_ref.at[base + 0], buf0, sem0),
        pltpu.async_copy(logits_ref.at[base + 1], buf1, sem1),
    ]
    for c in wc:
        c.wait()

    neg = jnp.full((L,), -jnp.inf, dtype=jnp.float32)
    lane = lax.iota(jnp.int32, L)
    T5 = [neg] * 5

    for j in range(RPW):
        b = bufs[j % 2]
        descs[j % 2].wait()

        def chunkmax(c, _, b=b):
            cb = c * (CVEC * L)
            m = b[pl.ds(cb, L)]
            for i in range(1, CVEC):
                m = jnp.maximum(m, b[pl.ds(cb + i * L, L)])
            cmbuf[pl.ds(c * L, L)] = m
            return 0

        lax.fori_loop(0, NCHUNK, chunkmax, 0, unroll=2)

        gms = []
        for g in range(NGRP):
            gm = None
            for l in range(CVEC):
                idx = g * (L * CVEC) + lane * L + l
                v = plsc.load_gather(cmbuf, [idx])
                gm = v if gm is None else jnp.maximum(gm, v)
            gms.append(gm)

        ms = [neg] * 5
        for gm in gms:
            ms = _insert5(ms, gm)
        for s in (1, 2, 4, 8):
            perm = jnp.bitwise_xor(lane, s)
            part = []
            for i in range(5):
                cand[...] = ms[i]
                part.append(plsc.load_gather(cand, [perm]))
            ms = _merge5(ms, part)
        tvec = ms[4]

        nvec = _splat(0)
        for g in range(NGRP):
            mask = gms[g] >= tvec
            ids = _splat(g * L) + lane
            pos = nvec + plsc.cumsum(mask.astype(jnp.int32)) - 1
            plsc.store_scatter(idxbuf, [pos], ids, mask=mask)
            nvec = nvec + plsc.all_reduce_population_count(mask)
        n = jnp.max(nvec)

        def scan_chunk(i, ms, b=b):
            cidv = plsc.load_gather(idxbuf, [_splat(0) + i])
            bv = cidv * (CVEC * L) + lane
            ms = list(ms)
            for i2 in range(CVEC):
                ms = _insert5(ms, plsc.load_gather(b, [bv + i2 * L]))
            return tuple(ms)

        ms = list(lax.fori_loop(0, n, scan_chunk, (neg,) * 5))

        if j + 2 < RPW:
            descs[j % 2] = pltpu.async_copy(
                logits_ref.at[base + j + 2], bufs[j % 2], sems[j % 2])

        for s in (1, 2, 4, 8):
            perm = jnp.bitwise_xor(lane, s)
            part = []
            for i in range(5):
                cand[...] = ms[i]
                part.append(plsc.load_gather(cand, [perm]))
            ms = _merge5(ms, part)

        is_j = lane == j
        for i in range(5):
            T5[i] = jnp.where(is_j, ms[i], T5[i])

    h = []
    for jo in range(5):
        acc = plsc.load_gather(b1v, [_splat(jo)])
        for i in range(5):
            acc = acc + plsc.load_gather(w1v, [_splat(jo * 5 + i)]) * T5[i]
        h.append(jnp.maximum(acc, 0.0))
    z = plsc.load_gather(b2v, [_splat(0)])
    for jo in range(5):
        z = z + plsc.load_gather(w2v, [_splat(jo)]) * h[jo]
    obuf[...] = 1.0 / (1.0 + jnp.exp(-z))
    pltpu.sync_copy(obuf, out_ref.at[wid])


@jax.jit
def _run(logits, W1, b1, W2, b2):
    mesh = plsc.VectorSubcoreMesh(
        core_axis_name="c", subcore_axis_name="s",
        num_cores=NC, num_subcores=NS)
    f = pl.kernel(
        _body,
        out_type=jax.ShapeDtypeStruct((NW, L), jnp.float32),
        mesh=mesh,
        compiler_params=pltpu.CompilerParams(needs_layout_passes=False),
        scratch_types=[
            pltpu.VMEM((COLS,), jnp.float32),
            pltpu.VMEM((COLS,), jnp.float32),
            pltpu.VMEM((NCHUNK * L,), jnp.float32),
            pltpu.VMEM((NCHUNK + L,), jnp.int32),
            pltpu.VMEM((L,), jnp.float32),
            pltpu.VMEM((25,), jnp.float32),
            pltpu.VMEM((5,), jnp.float32),
            pltpu.VMEM((5,), jnp.float32),
            pltpu.VMEM((1,), jnp.float32),
            pltpu.VMEM((L,), jnp.float32),
            pltpu.SemaphoreType.DMA,
            pltpu.SemaphoreType.DMA,
            pltpu.SemaphoreType.DMA,
        ],
    )
    return f(logits, W1, b1, W2, b2)


def kernel(logits, features, W1, b1, W2, b2):
    del features
    out32 = _run(logits, W1.reshape(-1), b1, W2.reshape(-1), b2)
    return out32[:, :RPW].reshape(ROWS, 1)


# speedup vs baseline: 1.1781x; 1.1781x over previous
import jax
import jax.numpy as jnp
from jax import lax
from jax.experimental import pallas as pl
from jax.experimental.pallas import tpu as pltpu
from jax.experimental.pallas import tpu_sc as plsc

NC, NS, L = 2, 16, 16
NW = NC * NS
ROWS, COLS = 128, 32768
RPW = ROWS // NW
NVEC = COLS // L
CVEC = 16
NCHUNK = NVEC // CVEC
NGRP = NCHUNK // L


def _treemax(vs):
    while len(vs) > 1:
        nxt = [jnp.maximum(vs[i], vs[i + 1]) for i in range(0, len(vs) - 1, 2)]
        if len(vs) % 2:
            nxt.append(vs[-1])
        vs = nxt
    return vs[0]


def _insert5(ms, v):
    out = []
    for i in range(4):
        hi = jnp.maximum(ms[i], v)
        v = jnp.minimum(ms[i], v)
        out.append(hi)
    out.append(jnp.maximum(ms[4], v))
    return out


def _merge5(a, b):
    r = []
    for k in range(5):
        cur = jnp.maximum(a[k], b[k])
        for i in range(k):
            cur = jnp.maximum(cur, jnp.minimum(a[i], b[k - 1 - i]))
        r.append(cur)
    return r


def _splat(val):
    return jnp.full((L,), val, dtype=jnp.int32)


def _body(logits_ref, wflat_ref, out_ref,
          bufd, cmbuf, gmbuf, idxbuf, cand, wv, obuf,
          sem0, sem1, wsem):
    wid = lax.axis_index("s") * NC + lax.axis_index("c")
    base = wid * RPW

    wd = pltpu.async_copy(wflat_ref, wv, wsem)
    pltpu.async_copy(logits_ref.at[base + 0], bufd.at[pl.ds(0, COLS)], sem0)
    pltpu.async_copy(logits_ref.at[base + 1], bufd.at[pl.ds(COLS, COLS)], sem1)
    wd.wait()

    neg = jnp.full((L,), -jnp.inf, dtype=jnp.float32)
    lane = lax.iota(jnp.int32, L)

    def process_row(off):
        def chunkmax(c, _):
            cb = off + c * (CVEC * L)
            m = _treemax([bufd[pl.ds(cb + i * L, L)] for i in range(CVEC)])
            cmbuf[pl.ds(c * L, L)] = m
            return 0

        lax.fori_loop(0, NCHUNK, chunkmax, 0, unroll=4)

        def grpmax(g, _):
            gm = _treemax([
                plsc.load_gather(cmbuf, [g * (L * CVEC) + lane * L + l])
                for l in range(CVEC)])
            gmbuf[pl.ds(g * L, L)] = gm
            return 0

        lax.fori_loop(0, NGRP, grpmax, 0)

        def ins(g, ms):
            return tuple(_insert5(list(ms), gmbuf[pl.ds(g * L, L)]))

        ms = list(lax.fori_loop(0, NGRP, ins, (neg,) * 5))
        for s in (1, 2, 4, 8):
            perm = jnp.bitwise_xor(lane, s)
            part = []
            for i in range(5):
                cand[...] = ms[i]
                part.append(plsc.load_gather(cand, [perm]))
            ms = _merge5(ms, part)
        tvec = ms[4]

        def compact(g, nvec):
            gm = gmbuf[pl.ds(g * L, L)]
            mask = gm >= tvec
            ids = g * L + lane
            pos = nvec + plsc.cumsum(mask.astype(jnp.int32)) - 1
            plsc.store_scatter(idxbuf, [pos], ids, mask=mask)
            return nvec + plsc.all_reduce_population_count(mask)

        nvec = lax.fori_loop(0, NGRP, compact, _splat(0))
        n = jnp.max(nvec)

        def scan_chunk(i, ms):
            cidv = plsc.load_gather(idxbuf, [_splat(0) + i])
            bv = off + cidv * (CVEC * L) + lane
            ms = list(ms)
            for i2 in range(CVEC):
                ms = _insert5(ms, plsc.load_gather(bufd, [bv + i2 * L]))
            return tuple(ms)

        ms = list(lax.fori_loop(0, n, scan_chunk, (neg,) * 5))

        for s in (1, 2, 4, 8):
            perm = jnp.bitwise_xor(lane, s)
            part = []
            for i in range(5):
                cand[...] = ms[i]
                part.append(plsc.load_gather(cand, [perm]))
            ms = _merge5(ms, part)
        return ms

    def row_body(j, T5):
        par = j % 2
        off = par * COLS

        @pl.when(par == 0)
        def _():
            pltpu.make_async_copy(
                logits_ref.at[base], bufd.at[pl.ds(0, COLS)], sem0).wait()

        @pl.when(par == 1)
        def _():
            pltpu.make_async_copy(
                logits_ref.at[base], bufd.at[pl.ds(COLS, COLS)], sem1).wait()

        ms = process_row(off)

        @pl.when(jnp.logical_and(j < RPW - 2, par == 0))
        def _():
            pltpu.async_copy(
                logits_ref.at[base + j + 2], bufd.at[pl.ds(0, COLS)], sem0)

        @pl.when(jnp.logical_and(j < RPW - 2, par == 1))
        def _():
            pltpu.async_copy(
                logits_ref.at[base + j + 2], bufd.at[pl.ds(COLS, COLS)], sem1)

        is_j = lane == j
        return tuple(jnp.where(is_j, ms[i], T5[i]) for i in range(5))

    T5 = list(lax.fori_loop(0, RPW, row_body, (neg,) * 5))

    def w(r):
        return plsc.load_gather(wv, [_splat(r)])

    h = []
    for jo in range(5):
        acc = w(25 + jo)
        for i in range(5):
            acc = acc + w(jo * 5 + i) * T5[i]
        h.append(jnp.maximum(acc, 0.0))
    z = w(35)
    for jo in range(5):
        z = z + w(30 + jo) * h[jo]
    obuf[...] = 1.0 / (1.0 + jnp.exp(-z))
    pltpu.sync_copy(obuf, out_ref.at[wid])


@jax.jit
def _run(logits, wflat):
    mesh = plsc.VectorSubcoreMesh(
        core_axis_name="c", subcore_axis_name="s",
        num_cores=NC, num_subcores=NS)
    f = pl.kernel(
        _body,
        out_type=jax.ShapeDtypeStruct((NW, L), jnp.float32),
        mesh=mesh,
        compiler_params=pltpu.CompilerParams(needs_layout_passes=False),
        scratch_types=[
            pltpu.VMEM((2 * COLS,), jnp.float32),
            pltpu.VMEM((NCHUNK * L,), jnp.float32),
            pltpu.VMEM((NGRP * L,), jnp.float32),
            pltpu.VMEM((NCHUNK + L,), jnp.int32),
            pltpu.VMEM((L,), jnp.float32),
            pltpu.VMEM((36,), jnp.float32),
            pltpu.VMEM((L,), jnp.float32),
            pltpu.SemaphoreType.DMA,
            pltpu.SemaphoreType.DMA,
            pltpu.SemaphoreType.DMA,
        ],
    )
    return f(logits, wflat)


def kernel(logits, features, W1, b1, W2, b2):
    del features
    wflat = jnp.concatenate(
        [W1.reshape(-1), b1.reshape(-1), W2.reshape(-1), b2.reshape(-1)])
    out32 = _run(logits, wflat)
    return out32[:, :RPW].reshape(ROWS, 1)


# speedup vs baseline: 1.1829x; 1.0040x over previous
import jax
import jax.numpy as jnp
from jax import lax
from jax.experimental import pallas as pl
from jax.experimental.pallas import tpu as pltpu
from jax.experimental.pallas import tpu_sc as plsc

NC, NS, L = 2, 16, 16
NW = NC * NS
ROWS, COLS = 128, 32768
RPW = ROWS // NW
NVEC = COLS // L
CVEC = 16
NCHUNK = NVEC // CVEC
NGRP = NCHUNK // L


def _treemax(vs):
    while len(vs) > 1:
        nxt = [jnp.maximum(vs[i], vs[i + 1]) for i in range(0, len(vs) - 1, 2)]
        if len(vs) % 2:
            nxt.append(vs[-1])
        vs = nxt
    return vs[0]


def _insert5(ms, v):
    out = []
    for i in range(4):
        hi = jnp.maximum(ms[i], v)
        v = jnp.minimum(ms[i], v)
        out.append(hi)
    out.append(jnp.maximum(ms[4], v))
    return out


def _merge5(a, b):
    r = []
    for k in range(5):
        cur = jnp.maximum(a[k], b[k])
        for i in range(k):
            cur = jnp.maximum(cur, jnp.minimum(a[i], b[k - 1 - i]))
        r.append(cur)
    return r


def _splat(val):
    return jnp.full((L,), val, dtype=jnp.int32)


def _body(logits_ref, wflat_ref, out_ref,
          bufd, cmbuf, gmbuf, idxbuf, cand, wv, obuf,
          sem0, sem1, wsem):
    wid = lax.axis_index("s") * NC + lax.axis_index("c")
    base = wid * RPW

    wd = pltpu.async_copy(wflat_ref, wv, wsem)
    pltpu.async_copy(logits_ref.at[base + 0], bufd.at[pl.ds(0, COLS)], sem0)
    pltpu.async_copy(logits_ref.at[base + 1], bufd.at[pl.ds(COLS, COLS)], sem1)
    wd.wait()

    neg = jnp.full((L,), -jnp.inf, dtype=jnp.float32)
    lane = lax.iota(jnp.int32, L)

    def process_row(off):
        def chunkmax(c, _):
            cb = off + c * (CVEC * L)
            m = _treemax([bufd[pl.ds(cb + i * L, L)] for i in range(CVEC)])
            cmbuf[pl.ds(c * L, L)] = m
            return 0

        with jax.named_scope("phaseA"):
            lax.fori_loop(0, NCHUNK, chunkmax, 0, unroll=4)

        def grpmax(g, _):
            gm = _treemax([
                plsc.load_gather(cmbuf, [g * (L * CVEC) + lane * L + l])
                for l in range(CVEC)])
            gmbuf[pl.ds(g * L, L)] = gm
            return 0

        with jax.named_scope("phaseBt"):
            lax.fori_loop(0, NGRP, grpmax, 0)

        def ins(g, ms):
            return tuple(_insert5(list(ms), gmbuf[pl.ds(g * L, L)]))

        with jax.named_scope("phaseBs"):
            ms = list(lax.fori_loop(0, NGRP, ins, (neg,) * 5))
            for s in (1, 2, 4, 8):
                perm = jnp.bitwise_xor(lane, s)
                part = []
                for i in range(5):
                    cand[...] = ms[i]
                    part.append(plsc.load_gather(cand, [perm]))
                ms = _merge5(ms, part)
            tvec = ms[4]

        def compact(g, nvec):
            gm = gmbuf[pl.ds(g * L, L)]
            mask = gm >= tvec
            ids = g * L + lane
            pos = nvec + plsc.cumsum(mask.astype(jnp.int32)) - 1
            plsc.store_scatter(idxbuf, [pos], ids, mask=mask)
            return nvec + plsc.all_reduce_population_count(mask)

        with jax.named_scope("phaseBc"):
            nvec = lax.fori_loop(0, NGRP, compact, _splat(0))
            n = jnp.max(nvec)

        def scan_chunk(i, ms):
            cidv = plsc.load_gather(idxbuf, [_splat(0) + i])
            bv = off + cidv * (CVEC * L) + lane
            ms = list(ms)
            for i2 in range(CVEC):
                ms = _insert5(ms, plsc.load_gather(bufd, [bv + i2 * L]))
            return tuple(ms)

        with jax.named_scope("phaseC"):
            ms = list(lax.fori_loop(0, n, scan_chunk, (neg,) * 5))

        with jax.named_scope("phaseM"):
            for s in (1, 2, 4, 8):
                perm = jnp.bitwise_xor(lane, s)
                part = []
                for i in range(5):
                    cand[...] = ms[i]
                    part.append(plsc.load_gather(cand, [perm]))
                ms = _merge5(ms, part)
        return ms

    def row_body(j, T5):
        par = j % 2
        off = par * COLS

        @pl.when(par == 0)
        def _():
            pltpu.make_async_copy(
                logits_ref.at[base], bufd.at[pl.ds(0, COLS)], sem0).wait()

        @pl.when(par == 1)
        def _():
            pltpu.make_async_copy(
                logits_ref.at[base], bufd.at[pl.ds(COLS, COLS)], sem1).wait()

        ms = process_row(off)

        @pl.when(jnp.logical_and(j < RPW - 2, par == 0))
        def _():
            pltpu.async_copy(
                logits_ref.at[base + j + 2], bufd.at[pl.ds(0, COLS)], sem0)

        @pl.when(jnp.logical_and(j < RPW - 2, par == 1))
        def _():
            pltpu.async_copy(
                logits_ref.at[base + j + 2], bufd.at[pl.ds(COLS, COLS)], sem1)

        is_j = lane == j
        return tuple(jnp.where(is_j, ms[i], T5[i]) for i in range(5))

    T5 = list(lax.fori_loop(0, RPW, row_body, (neg,) * 5))

    def w(r):
        return plsc.load_gather(wv, [_splat(r)])

    h = []
    for jo in range(5):
        acc = w(25 + jo)
        for i in range(5):
            acc = acc + w(jo * 5 + i) * T5[i]
        h.append(jnp.maximum(acc, 0.0))
    z = w(35)
    for jo in range(5):
        z = z + w(30 + jo) * h[jo]
    obuf[...] = 1.0 / (1.0 + jnp.exp(-z))
    pltpu.sync_copy(obuf, out_ref.at[wid])


@jax.jit
def _run(logits, wflat):
    mesh = plsc.VectorSubcoreMesh(
        core_axis_name="c", subcore_axis_name="s",
        num_cores=NC, num_subcores=NS)
    f = pl.kernel(
        _body,
        out_type=jax.ShapeDtypeStruct((NW, L), jnp.float32),
        mesh=mesh,
        compiler_params=pltpu.CompilerParams(needs_layout_passes=False),
        scratch_types=[
            pltpu.VMEM((2 * COLS,), jnp.float32),
            pltpu.VMEM((NCHUNK * L,), jnp.float32),
            pltpu.VMEM((NGRP * L,), jnp.float32),
            pltpu.VMEM((NCHUNK + L,), jnp.int32),
            pltpu.VMEM((L,), jnp.float32),
            pltpu.VMEM((36,), jnp.float32),
            pltpu.VMEM((L,), jnp.float32),
            pltpu.SemaphoreType.DMA,
            pltpu.SemaphoreType.DMA,
            pltpu.SemaphoreType.DMA,
        ],
    )
    return f(logits, wflat)


def kernel(logits, features, W1, b1, W2, b2):
    del features
    wflat = jnp.concatenate(
        [W1.reshape(-1), b1.reshape(-1), W2.reshape(-1), b2.reshape(-1)])
    out32 = _run(logits, wflat)
    return out32[:, :RPW].reshape(ROWS, 1)
